# baseline (speedup 1.0000x reference)
import functools

import jax
import jax.numpy as jnp
from jax import lax
from jax.experimental import pallas as pl
from jax.experimental.pallas import tpu as pltpu
from jax.experimental.pallas import tpu_sc as plsc

VOCAB = 30000
EMB = 128
D = 768
E = 8
K = 3
DFF = 768
T = 2048
CAP = 1024

TB1 = 256
NB = 2048
NEG = -1e30


_NW = 32
_BPW = T // _NW


def _sc_gather(table, idx):
    mesh = plsc.VectorSubcoreMesh(core_axis_name="c", subcore_axis_name="s")

    @functools.partial(
        pl.kernel,
        mesh=mesh,
        out_type=jax.ShapeDtypeStruct((T, EMB), jnp.float32),
        scratch_types=[
            pltpu.VMEM((_BPW,), jnp.int32),
            pltpu.VMEM((_BPW, EMB), jnp.float32),
            pltpu.SemaphoreType.DMA,
        ],
    )
    def k(table_hbm, idx_hbm, out_hbm, idx_v, rows_v, sem):
        wid = lax.axis_index("s") * 2 + lax.axis_index("c")
        base = wid * _BPW
        pltpu.sync_copy(idx_hbm.at[pl.ds(base, _BPW)], idx_v)
        pltpu.async_copy(table_hbm.at[idx_v], rows_v, sem).wait()
        pltpu.sync_copy(rows_v, out_hbm.at[pl.ds(base, _BPW)])

    return k(table, idx)


def _stage1_body(rows_ref, pos_ref, type_ref, g_ref, b_ref, win_ref, bin_ref,
                 wg_ref, x_ref, w_ref, cnt_ref):
    i = pl.program_id(0)

    @pl.when(i == 0)
    def _():
        cnt_ref[...] = jnp.zeros_like(cnt_ref)

    emb = rows_ref[...] + pos_ref[...] + type_ref[...]
    mu = jnp.mean(emb, axis=1, keepdims=True)
    d = emb - mu
    var = jnp.mean(d * d, axis=1, keepdims=True)
    ln = d * lax.rsqrt(var + 1e-12) * g_ref[...] + b_ref[...]

    x = jnp.dot(ln, win_ref[...], preferred_element_type=jnp.float32)
    x = x + bin_ref[...]
    x_ref[...] = x

    logits = jnp.dot(x, wg_ref[...], preferred_element_type=jnp.float32)
    lane = lax.broadcasted_iota(jnp.int32, (TB1, 128), 1)
    logits = jnp.where(lane < E, logits, NEG)

    cur = logits
    vals, ohs = [], []
    for _ in range(K):
        vk = jnp.max(cur, axis=1, keepdims=True)
        ik = jnp.min(jnp.where(cur == vk, lane, 128), axis=1, keepdims=True)
        oh = (lane == ik).astype(jnp.float32)
        vals.append(vk)
        ohs.append(oh)
        cur = jnp.where(lane == ik, NEG, cur)

    es = [jnp.exp(v - vals[0]) for v in vals]
    denom = es[0] + es[1] + es[2]
    gates = [e / denom for e in es]

    c = ohs[0] + ohs[1] + ohs[2]
    ri = lax.broadcasted_iota(jnp.int32, (TB1, TB1), 0)
    rj = lax.broadcasted_iota(jnp.int32, (TB1, TB1), 1)
    lt = (rj < ri).astype(jnp.float32)
    cex = jnp.dot(lt, c, preferred_element_type=jnp.float32)
    pos = cex + cnt_ref[...]

    w = jnp.zeros((TB1, 128), jnp.float32)
    for kk in range(K):
        pk = jnp.sum(ohs[kk] * pos, axis=1, keepdims=True)
        valid = (pk < CAP).astype(jnp.float32)
        w = w + gates[kk] * valid * ohs[kk]
    w_ref[...] = w
    cnt_ref[...] = cnt_ref[...] + jnp.sum(c, axis=0, keepdims=True)


def _stage1(rows, pos_emb, type_row, ln_g, ln_b, W_in, b_in, W_gate_p):
    nblk = T // TB1
    return pl.pallas_call(
        _stage1_body,
        grid=(nblk,),
        in_specs=[
            pl.BlockSpec((TB1, EMB), lambda i: (i, 0)),
            pl.BlockSpec((TB1, EMB), lambda i: (i, 0)),
            pl.BlockSpec((1, EMB), lambda i: (0, 0)),
            pl.BlockSpec((1, EMB), lambda i: (0, 0)),
            pl.BlockSpec((1, EMB), lambda i: (0, 0)),
            pl.BlockSpec((EMB, D), lambda i: (0, 0)),
            pl.BlockSpec((1, D), lambda i: (0, 0)),
            pl.BlockSpec((D, 128), lambda i: (0, 0)),
        ],
        out_specs=[
            pl.BlockSpec((TB1, D), lambda i: (i, 0)),
            pl.BlockSpec((TB1, 128), lambda i: (i, 0)),
        ],
        out_shape=[
            jax.ShapeDtypeStruct((T, D), jnp.float32),
            jax.ShapeDtypeStruct((T, 128), jnp.float32),
        ],
        scratch_shapes=[pltpu.VMEM((1, 128), jnp.float32)],
    )(rows, pos_emb, type_row, ln_g, ln_b, W_in, b_in, W_gate_p)


def _moe_body(x_ref, w_ref, w1_ref, b1_ref, w2_ref, b2_ref, out_ref):
    e = pl.program_id(0)
    x = x_ref[...]
    h = jnp.dot(x, w1_ref[0], preferred_element_type=jnp.float32) + b1_ref[0]
    h = jax.nn.gelu(h)
    y = jnp.dot(h, w2_ref[0], preferred_element_type=jnp.float32) + b2_ref[0]
    lane = lax.broadcasted_iota(jnp.int32, (T, 128), 1)
    we = jnp.sum(jnp.where(lane == e, w_ref[...], 0.0), axis=1, keepdims=True)
    contrib = we * y

    @pl.when(e == 0)
    def _():
        out_ref[...] = contrib

    @pl.when(e > 0)
    def _():
        out_ref[...] = out_ref[...] + contrib


def _moe(x, w, W1, b1, W2, b2):
    return pl.pallas_call(
        _moe_body,
        grid=(E,),
        in_specs=[
            pl.BlockSpec((T, D), lambda e: (0, 0)),
            pl.BlockSpec((T, 128), lambda e: (0, 0)),
            pl.BlockSpec((1, D, DFF), lambda e: (e, 0, 0)),
            pl.BlockSpec((1, 1, DFF), lambda e: (e, 0, 0)),
            pl.BlockSpec((1, DFF, D), lambda e: (e, 0, 0)),
            pl.BlockSpec((1, 1, D), lambda e: (e, 0, 0)),
        ],
        out_specs=pl.BlockSpec((T, D), lambda e: (0, 0)),
        out_shape=jax.ShapeDtypeStruct((T, D), jnp.float32),
    )(x, w, W1, b1.reshape(E, 1, DFF), W2, b2.reshape(E, 1, D))


def _lm_body(x_ref, wd_ref, bd_ref, g_ref, b_ref, hid_ref):
    h = jnp.dot(x_ref[...], wd_ref[...], preferred_element_type=jnp.float32)
    h = jax.nn.gelu(h + bd_ref[...])
    mu = jnp.mean(h, axis=1, keepdims=True)
    d = h - mu
    var = jnp.mean(d * d, axis=1, keepdims=True)
    hid_ref[...] = d * lax.rsqrt(var + 1e-12) * g_ref[...] + b_ref[...]


def _lm(x, lm_dense, lm_dense_b, lm_ln_g, lm_ln_b):
    return pl.pallas_call(
        _lm_body,
        out_shape=jax.ShapeDtypeStruct((T, EMB), jnp.float32),
    )(x, lm_dense, lm_dense_b.reshape(1, EMB),
      lm_ln_g.reshape(1, EMB), lm_ln_b.reshape(1, EMB))


def _dec_body(hid_ref, dec_ref, db_ref, out_ref):
    out_ref[...] = (
        jnp.dot(hid_ref[...], dec_ref[...], preferred_element_type=jnp.float32)
        + db_ref[...]
    )


def _dec_part(hid, dec_part, db_part):
    v = dec_part.shape[1]
    nblk = pl.cdiv(v, NB)
    return pl.pallas_call(
        _dec_body,
        grid=(nblk,),
        in_specs=[
            pl.BlockSpec((T, EMB), lambda j: (0, 0)),
            pl.BlockSpec((EMB, NB), lambda j: (0, j)),
            pl.BlockSpec((1, NB), lambda j: (0, j)),
        ],
        out_specs=pl.BlockSpec((T, NB), lambda j: (0, j)),
        out_shape=jax.ShapeDtypeStruct((T, v), jnp.float32),
    )(hid, dec_part, db_part)


_SPLITS = (7680, 15360, 23040)


def _dec(hid, decoder, decoder_b):
    db = decoder_b.reshape(1, VOCAB)
    bounds = (0,) + _SPLITS + (VOCAB,)
    out = jnp.zeros((1, T, VOCAB), jnp.float32)
    for lo, hi in zip(bounds[:-1], bounds[1:]):
        part = _dec_part(hid, decoder[:, lo:hi], db[:, lo:hi])
        out = lax.dynamic_update_slice(out, part.reshape(1, T, hi - lo),
                                       (0, 0, lo))
    return out


def kernel(input_ids, word_emb, pos_emb, type_emb, ln_emb_g, ln_emb_b, W_in,
           b_in, W_gate, W1, b1, W2, b2, lm_dense, lm_dense_b, lm_ln_g,
           lm_ln_b, decoder, decoder_b):
    idx = input_ids.reshape(T).astype(jnp.int32)
    rows = _sc_gather(word_emb, idx)

    W_gate_p = jnp.pad(W_gate, ((0, 0), (0, 128 - E)))
    x, w = _stage1(rows, pos_emb, type_emb[0:1, :], ln_emb_g.reshape(1, EMB),
                   ln_emb_b.reshape(1, EMB), W_in, b_in.reshape(1, D), W_gate_p)

    xm = _moe(x, w, W1, b1, W2, b2)
    hid = _lm(xm, lm_dense, lm_dense_b, lm_ln_g, lm_ln_b)
    scores = _dec(hid, decoder, decoder_b)
    return scores, xm.reshape(1, T, D)


# speedup vs baseline: 1.5098x; 1.5098x over previous
import functools

import jax
import jax.numpy as jnp
from jax import lax
from jax.experimental import pallas as pl
from jax.experimental.pallas import tpu as pltpu
from jax.experimental.pallas import tpu_sc as plsc

VOCAB = 30000
EMB = 128
D = 768
E = 8
K = 3
DFF = 768
T = 2048
CAP = 1024

TB1 = 256
NB = 2048
NEG = -1e30


_NW = 32
_BPW = T // _NW


def _sc_gather(table, idx):
    mesh = plsc.VectorSubcoreMesh(core_axis_name="c", subcore_axis_name="s")

    @functools.partial(
        pl.kernel,
        mesh=mesh,
        out_type=jax.ShapeDtypeStruct((T, EMB), jnp.float32),
        scratch_types=[
            pltpu.VMEM((_BPW,), jnp.int32),
            pltpu.VMEM((_BPW, EMB), jnp.float32),
            pltpu.SemaphoreType.DMA,
        ],
    )
    def k(table_hbm, idx_hbm, out_hbm, idx_v, rows_v, sem):
        wid = lax.axis_index("s") * 2 + lax.axis_index("c")
        base = wid * _BPW
        pltpu.sync_copy(idx_hbm.at[pl.ds(base, _BPW)], idx_v)
        pltpu.async_copy(table_hbm.at[idx_v], rows_v, sem).wait()
        pltpu.sync_copy(rows_v, out_hbm.at[pl.ds(base, _BPW)])

    return k(table, idx)


def _stage1_body(rows_ref, pos_ref, type_ref, g_ref, b_ref, win_ref, bin_ref,
                 wg_ref, x_ref, w_ref, cnt_ref):
    i = pl.program_id(0)

    @pl.when(i == 0)
    def _():
        cnt_ref[...] = jnp.zeros_like(cnt_ref)

    emb = rows_ref[...] + pos_ref[...] + type_ref[...]
    mu = jnp.mean(emb, axis=1, keepdims=True)
    d = emb - mu
    var = jnp.mean(d * d, axis=1, keepdims=True)
    ln = d * lax.rsqrt(var + 1e-12) * g_ref[...] + b_ref[...]

    x = jnp.dot(ln, win_ref[...], preferred_element_type=jnp.float32)
    x = x + bin_ref[...]
    x_ref[...] = x

    logits = jnp.dot(x, wg_ref[...], preferred_element_type=jnp.float32)
    lane = lax.broadcasted_iota(jnp.int32, (TB1, 128), 1)
    logits = jnp.where(lane < E, logits, NEG)

    cur = logits
    vals, ohs = [], []
    for _ in range(K):
        vk = jnp.max(cur, axis=1, keepdims=True)
        ik = jnp.min(jnp.where(cur == vk, lane, 128), axis=1, keepdims=True)
        oh = (lane == ik).astype(jnp.float32)
        vals.append(vk)
        ohs.append(oh)
        cur = jnp.where(lane == ik, NEG, cur)

    es = [jnp.exp(v - vals[0]) for v in vals]
    denom = es[0] + es[1] + es[2]
    gates = [e / denom for e in es]

    c = ohs[0] + ohs[1] + ohs[2]
    ri = lax.broadcasted_iota(jnp.int32, (TB1, TB1), 0)
    rj = lax.broadcasted_iota(jnp.int32, (TB1, TB1), 1)
    lt = (rj < ri).astype(jnp.float32)
    cex = jnp.dot(lt, c, preferred_element_type=jnp.float32)
    pos = cex + cnt_ref[...]

    w = jnp.zeros((TB1, 128), jnp.float32)
    for kk in range(K):
        pk = jnp.sum(ohs[kk] * pos, axis=1, keepdims=True)
        valid = (pk < CAP).astype(jnp.float32)
        w = w + gates[kk] * valid * ohs[kk]
    w_ref[...] = w
    cnt_ref[...] = cnt_ref[...] + jnp.sum(c, axis=0, keepdims=True)


def _stage1(rows, pos_emb, type_row, ln_g, ln_b, W_in, b_in, W_gate_p):
    nblk = T // TB1
    return pl.pallas_call(
        _stage1_body,
        grid=(nblk,),
        in_specs=[
            pl.BlockSpec((TB1, EMB), lambda i: (i, 0)),
            pl.BlockSpec((TB1, EMB), lambda i: (i, 0)),
            pl.BlockSpec((1, EMB), lambda i: (0, 0)),
            pl.BlockSpec((1, EMB), lambda i: (0, 0)),
            pl.BlockSpec((1, EMB), lambda i: (0, 0)),
            pl.BlockSpec((EMB, D), lambda i: (0, 0)),
            pl.BlockSpec((1, D), lambda i: (0, 0)),
            pl.BlockSpec((D, 128), lambda i: (0, 0)),
        ],
        out_specs=[
            pl.BlockSpec((TB1, D), lambda i: (i, 0)),
            pl.BlockSpec((TB1, 128), lambda i: (i, 0)),
        ],
        out_shape=[
            jax.ShapeDtypeStruct((T, D), jnp.float32),
            jax.ShapeDtypeStruct((T, 128), jnp.float32),
        ],
        scratch_shapes=[pltpu.VMEM((1, 128), jnp.float32)],
    )(rows, pos_emb, type_row, ln_g, ln_b, W_in, b_in, W_gate_p)


def _moe_body(x_ref, w_ref, w1_ref, b1_ref, w2_ref, b2_ref, out_ref):
    e = pl.program_id(0)
    x = x_ref[...]
    h = jnp.dot(x, w1_ref[0], preferred_element_type=jnp.float32) + b1_ref[0]
    h = jax.nn.gelu(h)
    y = jnp.dot(h, w2_ref[0], preferred_element_type=jnp.float32) + b2_ref[0]
    lane = lax.broadcasted_iota(jnp.int32, (T, 128), 1)
    we = jnp.sum(jnp.where(lane == e, w_ref[...], 0.0), axis=1, keepdims=True)
    contrib = we * y

    @pl.when(e == 0)
    def _():
        out_ref[...] = contrib

    @pl.when(e > 0)
    def _():
        out_ref[...] = out_ref[...] + contrib


def _moe(x, w, W1, b1, W2, b2):
    return pl.pallas_call(
        _moe_body,
        grid=(E,),
        in_specs=[
            pl.BlockSpec((T, D), lambda e: (0, 0)),
            pl.BlockSpec((T, 128), lambda e: (0, 0)),
            pl.BlockSpec((1, D, DFF), lambda e: (e, 0, 0)),
            pl.BlockSpec((1, 1, DFF), lambda e: (e, 0, 0)),
            pl.BlockSpec((1, DFF, D), lambda e: (e, 0, 0)),
            pl.BlockSpec((1, 1, D), lambda e: (e, 0, 0)),
        ],
        out_specs=pl.BlockSpec((T, D), lambda e: (0, 0)),
        out_shape=jax.ShapeDtypeStruct((T, D), jnp.float32),
    )(x, w, W1, b1.reshape(E, 1, DFF), W2, b2.reshape(E, 1, D))


def _lm_body(x_ref, wd_ref, bd_ref, g_ref, b_ref, hid_ref):
    h = jnp.dot(x_ref[...], wd_ref[...], preferred_element_type=jnp.float32)
    h = jax.nn.gelu(h + bd_ref[...])
    mu = jnp.mean(h, axis=1, keepdims=True)
    d = h - mu
    var = jnp.mean(d * d, axis=1, keepdims=True)
    hid_ref[...] = d * lax.rsqrt(var + 1e-12) * g_ref[...] + b_ref[...]


def _lm(x, lm_dense, lm_dense_b, lm_ln_g, lm_ln_b):
    return pl.pallas_call(
        _lm_body,
        out_shape=jax.ShapeDtypeStruct((T, EMB), jnp.float32),
    )(x, lm_dense, lm_dense_b.reshape(1, EMB),
      lm_ln_g.reshape(1, EMB), lm_ln_b.reshape(1, EMB))


def _dec_body(hid_ref, dec_ref, db_ref, out_ref):
    out_ref[...] = (
        jnp.dot(hid_ref[...], dec_ref[...], preferred_element_type=jnp.float32)
        + db_ref[...]
    )


TBD = 128


def _dec(hid, decoder, decoder_b):
    return pl.pallas_call(
        _dec_body,
        grid=(T // TBD,),
        in_specs=[
            pl.BlockSpec((TBD, EMB), lambda i: (i, 0)),
            pl.BlockSpec((EMB, VOCAB), lambda i: (0, 0)),
            pl.BlockSpec((1, VOCAB), lambda i: (0, 0)),
        ],
        out_specs=pl.BlockSpec((None, TBD, VOCAB), lambda i: (0, i, 0)),
        out_shape=jax.ShapeDtypeStruct((1, T, VOCAB), jnp.float32),
    )(hid, decoder, decoder_b.reshape(1, VOCAB))


def kernel(input_ids, word_emb, pos_emb, type_emb, ln_emb_g, ln_emb_b, W_in,
           b_in, W_gate, W1, b1, W2, b2, lm_dense, lm_dense_b, lm_ln_g,
           lm_ln_b, decoder, decoder_b):
    idx = input_ids.reshape(T).astype(jnp.int32)
    rows = _sc_gather(word_emb, idx)

    W_gate_p = jnp.pad(W_gate, ((0, 0), (0, 128 - E)))
    x, w = _stage1(rows, pos_emb, type_emb[0:1, :], ln_emb_g.reshape(1, EMB),
                   ln_emb_b.reshape(1, EMB), W_in, b_in.reshape(1, D), W_gate_p)

    xm = _moe(x, w, W1, b1, W2, b2)
    hid = _lm(xm, lm_dense, lm_dense_b, lm_ln_g, lm_ln_b)
    scores = _dec(hid, decoder, decoder_b)
    return scores, xm.reshape(1, T, D)


# speedup vs baseline: 1.5763x; 1.0440x over previous
import functools

import jax
import jax.numpy as jnp
from jax import lax
from jax.experimental import pallas as pl
from jax.experimental.pallas import tpu as pltpu
from jax.experimental.pallas import tpu_sc as plsc

VOCAB = 30000
EMB = 128
D = 768
E = 8
K = 3
DFF = 768
T = 2048
CAP = 1024

TB1 = 256
NB = 2048
NEG = -1e30


_NW = 32
_BPW = T // _NW


def _sc_gather(table, idx):
    mesh = plsc.VectorSubcoreMesh(core_axis_name="c", subcore_axis_name="s")

    @functools.partial(
        pl.kernel,
        mesh=mesh,
        out_type=jax.ShapeDtypeStruct((T, EMB), jnp.float32),
        scratch_types=[
            pltpu.VMEM((_BPW,), jnp.int32),
            pltpu.VMEM((_BPW, EMB), jnp.float32),
            pltpu.SemaphoreType.DMA,
        ],
    )
    def k(table_hbm, idx_hbm, out_hbm, idx_v, rows_v, sem):
        wid = lax.axis_index("s") * 2 + lax.axis_index("c")
        base = wid * _BPW
        pltpu.sync_copy(idx_hbm.at[pl.ds(base, _BPW)], idx_v)
        pltpu.async_copy(table_hbm.at[idx_v], rows_v, sem).wait()
        pltpu.sync_copy(rows_v, out_hbm.at[pl.ds(base, _BPW)])

    return k(table, idx)


def _stage1_body(rows_ref, pos_ref, type_ref, g_ref, b_ref, win_ref, bin_ref,
                 wg_ref, x_ref, w_ref, cnt_ref):
    i = pl.program_id(0)

    @pl.when(i == 0)
    def _():
        cnt_ref[...] = jnp.zeros_like(cnt_ref)

    emb = rows_ref[...] + pos_ref[...] + type_ref[...]
    mu = jnp.mean(emb, axis=1, keepdims=True)
    d = emb - mu
    var = jnp.mean(d * d, axis=1, keepdims=True)
    ln = d * lax.rsqrt(var + 1e-12) * g_ref[...] + b_ref[...]

    x = jnp.dot(ln, win_ref[...], preferred_element_type=jnp.float32)
    x = x + bin_ref[...]
    x_ref[...] = x

    logits = jnp.dot(x, wg_ref[...], preferred_element_type=jnp.float32)
    lane = lax.broadcasted_iota(jnp.int32, (TB1, 128), 1)
    logits = jnp.where(lane < E, logits, NEG)

    cur = logits
    vals, ohs = [], []
    for _ in range(K):
        vk = jnp.max(cur, axis=1, keepdims=True)
        ik = jnp.min(jnp.where(cur == vk, lane, 128), axis=1, keepdims=True)
        oh = (lane == ik).astype(jnp.float32)
        vals.append(vk)
        ohs.append(oh)
        cur = jnp.where(lane == ik, NEG, cur)

    es = [jnp.exp(v - vals[0]) for v in vals]
    denom = es[0] + es[1] + es[2]
    gates = [e / denom for e in es]

    c = ohs[0] + ohs[1] + ohs[2]
    ri = lax.broadcasted_iota(jnp.int32, (TB1, TB1), 0)
    rj = lax.broadcasted_iota(jnp.int32, (TB1, TB1), 1)
    lt = (rj < ri).astype(jnp.float32)
    cex = jnp.dot(lt, c, preferred_element_type=jnp.float32)
    pos = cex + cnt_ref[...]

    w = jnp.zeros((TB1, 128), jnp.float32)
    for kk in range(K):
        pk = jnp.sum(ohs[kk] * pos, axis=1, keepdims=True)
        valid = (pk < CAP).astype(jnp.float32)
        w = w + gates[kk] * valid * ohs[kk]
    w_ref[...] = w
    cnt_ref[...] = cnt_ref[...] + jnp.sum(c, axis=0, keepdims=True)


def _stage1(rows, pos_emb, type_row, ln_g, ln_b, W_in, b_in, W_gate_p):
    nblk = T // TB1
    return pl.pallas_call(
        _stage1_body,
        grid=(nblk,),
        in_specs=[
            pl.BlockSpec((TB1, EMB), lambda i: (i, 0)),
            pl.BlockSpec((TB1, EMB), lambda i: (i, 0)),
            pl.BlockSpec((1, EMB), lambda i: (0, 0)),
            pl.BlockSpec((1, EMB), lambda i: (0, 0)),
            pl.BlockSpec((1, EMB), lambda i: (0, 0)),
            pl.BlockSpec((EMB, D), lambda i: (0, 0)),
            pl.BlockSpec((1, D), lambda i: (0, 0)),
            pl.BlockSpec((D, 128), lambda i: (0, 0)),
        ],
        out_specs=[
            pl.BlockSpec((TB1, D), lambda i: (i, 0)),
            pl.BlockSpec((TB1, 128), lambda i: (i, 0)),
        ],
        out_shape=[
            jax.ShapeDtypeStruct((T, D), jnp.float32),
            jax.ShapeDtypeStruct((T, 128), jnp.float32),
        ],
        scratch_shapes=[pltpu.VMEM((1, 128), jnp.float32)],
    )(rows, pos_emb, type_row, ln_g, ln_b, W_in, b_in, W_gate_p)


def _moe_body(x_ref, w_ref, w1_ref, b1_ref, w2_ref, b2_ref, out_ref):
    e = pl.program_id(0)
    x = x_ref[...].astype(jnp.bfloat16)
    h = jnp.dot(x, w1_ref[0], preferred_element_type=jnp.float32) + b1_ref[0]
    h = jax.nn.gelu(h).astype(jnp.bfloat16)
    y = jnp.dot(h, w2_ref[0], preferred_element_type=jnp.float32) + b2_ref[0]
    lane = lax.broadcasted_iota(jnp.int32, (T, 128), 1)
    we = jnp.sum(jnp.where(lane == e, w_ref[...], 0.0), axis=1, keepdims=True)
    contrib = we * y

    @pl.when(e == 0)
    def _():
        out_ref[...] = contrib

    @pl.when(e > 0)
    def _():
        out_ref[...] = out_ref[...] + contrib


def _moe(x, w, W1, b1, W2, b2):
    return pl.pallas_call(
        _moe_body,
        grid=(E,),
        in_specs=[
            pl.BlockSpec((T, D), lambda e: (0, 0)),
            pl.BlockSpec((T, 128), lambda e: (0, 0)),
            pl.BlockSpec((1, D, DFF), lambda e: (e, 0, 0)),
            pl.BlockSpec((1, 1, DFF), lambda e: (e, 0, 0)),
            pl.BlockSpec((1, DFF, D), lambda e: (e, 0, 0)),
            pl.BlockSpec((1, 1, D), lambda e: (e, 0, 0)),
        ],
        out_specs=pl.BlockSpec((T, D), lambda e: (0, 0)),
        out_shape=jax.ShapeDtypeStruct((T, D), jnp.float32),
    )(x, w, W1.astype(jnp.bfloat16), b1.reshape(E, 1, DFF),
      W2.astype(jnp.bfloat16), b2.reshape(E, 1, D))


def _lm_body(x_ref, wd_ref, bd_ref, g_ref, b_ref, hid_ref):
    h = jnp.dot(x_ref[...], wd_ref[...], preferred_element_type=jnp.float32)
    h = jax.nn.gelu(h + bd_ref[...])
    mu = jnp.mean(h, axis=1, keepdims=True)
    d = h - mu
    var = jnp.mean(d * d, axis=1, keepdims=True)
    hid_ref[...] = d * lax.rsqrt(var + 1e-12) * g_ref[...] + b_ref[...]


def _lm(x, lm_dense, lm_dense_b, lm_ln_g, lm_ln_b):
    return pl.pallas_call(
        _lm_body,
        out_shape=jax.ShapeDtypeStruct((T, EMB), jnp.float32),
    )(x, lm_dense, lm_dense_b.reshape(1, EMB),
      lm_ln_g.reshape(1, EMB), lm_ln_b.reshape(1, EMB))


def _dec_body(hid_ref, dec_ref, db_ref, out_ref):
    out_ref[...] = (
        jnp.dot(hid_ref[...], dec_ref[...], preferred_element_type=jnp.float32)
        + db_ref[...]
    )


def _dec(hid, decoder, decoder_b):
    nblk = pl.cdiv(VOCAB, NB)
    return pl.pallas_call(
        _dec_body,
        grid=(nblk,),
        in_specs=[
            pl.BlockSpec((T, EMB), lambda j: (0, 0)),
            pl.BlockSpec((EMB, NB), lambda j: (0, j)),
            pl.BlockSpec((1, NB), lambda j: (0, j)),
        ],
        out_specs=pl.BlockSpec((T, NB), lambda j: (0, j)),
        out_shape=jax.ShapeDtypeStruct((T, VOCAB), jnp.float32),
    )(hid, decoder, decoder_b.reshape(1, VOCAB))


def kernel(input_ids, word_emb, pos_emb, type_emb, ln_emb_g, ln_emb_b, W_in,
           b_in, W_gate, W1, b1, W2, b2, lm_dense, lm_dense_b, lm_ln_g,
           lm_ln_b, decoder, decoder_b):
    idx = input_ids.reshape(T).astype(jnp.int32)
    rows = _sc_gather(word_emb, idx)

    W_gate_p = jnp.pad(W_gate, ((0, 0), (0, 128 - E)))
    x, w = _stage1(rows, pos_emb, type_emb[0:1, :], ln_emb_g.reshape(1, EMB),
                   ln_emb_b.reshape(1, EMB), W_in, b_in.reshape(1, D), W_gate_p)

    xm = _moe(x, w, W1, b1, W2, b2)
    hid = _lm(xm, lm_dense, lm_dense_b, lm_ln_g, lm_ln_b)
    scores = _dec(hid, decoder, decoder_b)
    return scores.reshape(1, T, VOCAB), xm.reshape(1, T, D)


# speedup vs baseline: 1.7226x; 1.0928x over previous
import functools

import jax
import jax.numpy as jnp
from jax import lax
from jax.experimental import pallas as pl
from jax.experimental.pallas import tpu as pltpu
from jax.experimental.pallas import tpu_sc as plsc

VOCAB = 30000
EMB = 128
D = 768
E = 8
K = 3
DFF = 768
T = 2048
CAP = 1024

TB1 = 256
NB = 2048
NEG = -1e30


_NW = 32
_BPW = T // _NW


def _sc_gather(table, idx):
    mesh = plsc.VectorSubcoreMesh(core_axis_name="c", subcore_axis_name="s")

    @functools.partial(
        pl.kernel,
        mesh=mesh,
        out_type=jax.ShapeDtypeStruct((T, EMB), jnp.float32),
        scratch_types=[
            pltpu.VMEM((_BPW,), jnp.int32),
            pltpu.VMEM((_BPW, EMB), jnp.float32),
            pltpu.SemaphoreType.DMA,
        ],
    )
    def k(table_hbm, idx_hbm, out_hbm, idx_v, rows_v, sem):
        wid = lax.axis_index("s") * 2 + lax.axis_index("c")
        base = wid * _BPW
        pltpu.sync_copy(idx_hbm.at[pl.ds(base, _BPW)], idx_v)
        pltpu.async_copy(table_hbm.at[idx_v], rows_v, sem).wait()
        pltpu.sync_copy(rows_v, out_hbm.at[pl.ds(base, _BPW)])

    return k(table, idx)


def _core_body(rows_ref, pos_ref, type_ref, g_ref, b_ref, win_ref, bin_ref,
               wg_ref, w1_ref, b1_ref, w2_ref, b2_ref, lmw_ref, lmb_ref,
               lmg_ref, lmbb_ref, xm_ref, hid_ref, xs_ref, ws_ref, acc_ref):
    e = pl.program_id(0)

    @pl.when(e == 0)
    def _():
        emb = rows_ref[...] + pos_ref[...] + type_ref[...]
        mu = jnp.mean(emb, axis=1, keepdims=True)
        d = emb - mu
        var = jnp.mean(d * d, axis=1, keepdims=True)
        ln = d * lax.rsqrt(var + 1e-12) * g_ref[...] + b_ref[...]

        x = jnp.dot(ln, win_ref[...], preferred_element_type=jnp.float32)
        x = x + bin_ref[...]
        xs_ref[...] = x

        logits = jnp.dot(x, wg_ref[...], preferred_element_type=jnp.float32)
        lane = lax.broadcasted_iota(jnp.int32, (T, 128), 1)
        logits = jnp.where(lane < E, logits, NEG)

        cur = logits
        vals, ohs = [], []
        for _ in range(K):
            vk = jnp.max(cur, axis=1, keepdims=True)
            ik = jnp.min(jnp.where(cur == vk, lane, 128), axis=1, keepdims=True)
            oh = (lane == ik).astype(jnp.float32)
            vals.append(vk)
            ohs.append(oh)
            cur = jnp.where(lane == ik, NEG, cur)

        es = [jnp.exp(v - vals[0]) for v in vals]
        denom = es[0] + es[1] + es[2]
        gates = [ex / denom for ex in es]

        c = (ohs[0] + ohs[1] + ohs[2]).astype(jnp.bfloat16)
        ri = lax.broadcasted_iota(jnp.int32, (T, T), 0)
        rj = lax.broadcasted_iota(jnp.int32, (T, T), 1)
        lt = (rj < ri).astype(jnp.bfloat16)
        pos = jnp.dot(lt, c, preferred_element_type=jnp.float32)

        w = jnp.zeros((T, 128), jnp.float32)
        for kk in range(K):
            pk = jnp.sum(ohs[kk] * pos, axis=1, keepdims=True)
            valid = (pk < CAP).astype(jnp.float32)
            w = w + gates[kk] * valid * ohs[kk]
        ws_ref[...] = w

    x = xs_ref[...]
    h = jnp.dot(x, w1_ref[0], preferred_element_type=jnp.float32) + b1_ref[0]
    h = jax.nn.gelu(h)
    y = jnp.dot(h, w2_ref[0], preferred_element_type=jnp.float32) + b2_ref[0]
    lane2 = lax.broadcasted_iota(jnp.int32, (T, 128), 1)
    we = jnp.sum(jnp.where(lane2 == e, ws_ref[...], 0.0), axis=1, keepdims=True)
    contrib = we * y

    @pl.when(e == 0)
    def _():
        acc_ref[...] = contrib

    @pl.when(e > 0)
    def _():
        acc_ref[...] = acc_ref[...] + contrib

    @pl.when(e == E - 1)
    def _():
        xm = acc_ref[...]
        xm_ref[...] = xm
        hh = jnp.dot(xm, lmw_ref[...], preferred_element_type=jnp.float32)
        hh = jax.nn.gelu(hh + lmb_ref[...])
        mu = jnp.mean(hh, axis=1, keepdims=True)
        d = hh - mu
        var = jnp.mean(d * d, axis=1, keepdims=True)
        hid_ref[...] = d * lax.rsqrt(var + 1e-12) * lmg_ref[...] + lmbb_ref[...]


def _core(rows, pos_emb, type_row, ln_g, ln_b, W_in, b_in, W_gate_p,
          W1, b1, W2, b2, lm_dense, lm_dense_b, lm_ln_g, lm_ln_b):
    full2 = lambda e: (0, 0)
    return pl.pallas_call(
        _core_body,
        grid=(E,),
        in_specs=[
            pl.BlockSpec((T, EMB), full2),
            pl.BlockSpec((T, EMB), full2),
            pl.BlockSpec((1, EMB), full2),
            pl.BlockSpec((1, EMB), full2),
            pl.BlockSpec((1, EMB), full2),
            pl.BlockSpec((EMB, D), full2),
            pl.BlockSpec((1, D), full2),
            pl.BlockSpec((D, 128), full2),
            pl.BlockSpec((1, D, DFF), lambda e: (e, 0, 0)),
            pl.BlockSpec((1, 1, DFF), lambda e: (e, 0, 0)),
            pl.BlockSpec((1, DFF, D), lambda e: (e, 0, 0)),
            pl.BlockSpec((1, 1, D), lambda e: (e, 0, 0)),
            pl.BlockSpec((D, EMB), full2),
            pl.BlockSpec((1, EMB), full2),
            pl.BlockSpec((1, EMB), full2),
            pl.BlockSpec((1, EMB), full2),
        ],
        out_specs=[
            pl.BlockSpec((T, D), full2),
            pl.BlockSpec((T, EMB), full2),
        ],
        out_shape=[
            jax.ShapeDtypeStruct((T, D), jnp.float32),
            jax.ShapeDtypeStruct((T, EMB), jnp.float32),
        ],
        scratch_shapes=[
            pltpu.VMEM((T, D), jnp.float32),
            pltpu.VMEM((T, 128), jnp.float32),
            pltpu.VMEM((T, D), jnp.float32),
        ],
    )(rows, pos_emb, type_row, ln_g, ln_b, W_in, b_in, W_gate_p,
      W1, b1.reshape(E, 1, DFF), W2, b2.reshape(E, 1, D),
      lm_dense, lm_dense_b.reshape(1, EMB), lm_ln_g.reshape(1, EMB),
      lm_ln_b.reshape(1, EMB))


def _dec_body(hid_ref, dec_ref, db_ref, out_ref):
    out_ref[...] = (
        jnp.dot(hid_ref[...], dec_ref[...], preferred_element_type=jnp.float32)
        + db_ref[...]
    )


def _dec(hid, decoder, decoder_b):
    nblk = pl.cdiv(VOCAB, NB)
    return pl.pallas_call(
        _dec_body,
        grid=(nblk,),
        in_specs=[
            pl.BlockSpec((T, EMB), lambda j: (0, 0)),
            pl.BlockSpec((EMB, NB), lambda j: (0, j)),
            pl.BlockSpec((1, NB), lambda j: (0, j)),
        ],
        out_specs=pl.BlockSpec((T, NB), lambda j: (0, j)),
        out_shape=jax.ShapeDtypeStruct((T, VOCAB), jnp.float32),
    )(hid, decoder, decoder_b.reshape(1, VOCAB))


def kernel(input_ids, word_emb, pos_emb, type_emb, ln_emb_g, ln_emb_b, W_in,
           b_in, W_gate, W1, b1, W2, b2, lm_dense, lm_dense_b, lm_ln_g,
           lm_ln_b, decoder, decoder_b):
    idx = input_ids.reshape(T).astype(jnp.int32)
    rows = _sc_gather(word_emb, idx)

    W_gate_p = jnp.pad(W_gate, ((0, 0), (0, 128 - E)))
    xm, hid = _core(rows, pos_emb, type_emb[0:1, :], ln_emb_g.reshape(1, EMB),
                    ln_emb_b.reshape(1, EMB), W_in, b_in.reshape(1, D),
                    W_gate_p, W1, b1, W2, b2, lm_dense, lm_dense_b,
                    lm_ln_g, lm_ln_b)
    scores = _dec(hid, decoder, decoder_b)
    return scores.reshape(1, T, VOCAB), xm.reshape(1, T, D)
